# baseline (speedup 1.0000x reference)
import functools

import jax
import jax.numpy as jnp
from jax import lax
from jax.experimental import pallas as pl
from jax.experimental.pallas import tpu as pltpu
from jax.experimental.pallas import tpu_sc as plsc

NUM_Q = 8
CODEBOOK_SIZE = 1024
DIM = 64
HIDDEN = 1024
BATCH = 16
TLEN = 2048
NTOK = BATCH * TLEN
KDIM = NUM_Q * DIM

NC = 2
NS = 16
NW = NC * NS
TOK_PER_W = NTOK // NW
CHUNK = 64
NCHUNK = TOK_PER_W // CHUNK


def _sc_gather(codes_flat, cb_flat):
    mesh = plsc.VectorSubcoreMesh(
        core_axis_name="c", subcore_axis_name="s", num_cores=NC, num_subcores=NS
    )

    @functools.partial(
        pl.kernel,
        mesh=mesh,
        out_type=jax.ShapeDtypeStruct((NTOK, KDIM), jnp.bfloat16),
        scratch_types=[
            pltpu.VMEM((NUM_Q, CHUNK), jnp.int32),
            pltpu.VMEM((NUM_Q, CHUNK, DIM), jnp.bfloat16),
            pltpu.SemaphoreType.DMA,
        ],
        compiler_params=pltpu.CompilerParams(use_tc_tiling_on_sc=False),
    )
    def k(codes_hbm, cb_hbm, q_hbm, idx_v, dst_v, sem):
        wid = lax.axis_index("s") * NC + lax.axis_index("c")
        wbase = wid * TOK_PER_W

        def chunk_body(ci, carry):
            base = wbase + ci * CHUNK
            pltpu.sync_copy(codes_hbm.at[:, pl.ds(base, CHUNK)], idx_v)
            for i in range(1, NUM_Q):
                for j in range(CHUNK // 16):
                    sl = pl.ds(j * 16, 16)
                    idx_v[i, sl] = idx_v[i, sl] + (i * CODEBOOK_SIZE)
            copies = [
                pltpu.async_copy(
                    cb_hbm.at[idx_v.at[i]],
                    dst_v.at[i],
                    sem,
                )
                for i in range(NUM_Q)
            ]
            for cp in copies:
                cp.wait()
            for i in range(NUM_Q):
                pltpu.sync_copy(
                    dst_v.at[i],
                    q_hbm.at[pl.ds(base, CHUNK), pl.ds(i * DIM, DIM)],
                )
            return carry

        lax.fori_loop(0, NCHUNK, chunk_body, 0)

    return k(codes_flat, cb_flat)


TB = 512


def _tc_matmul_body(q_ref, wt_ref, bt_ref, out_ref):
    qb = q_ref[...]
    acc = lax.dot_general(
        wt_ref[...], qb,
        dimension_numbers=(((1,), (1,)), ((), ())),
        preferred_element_type=jnp.float32,
    )
    bsum = jnp.sum(bt_ref[...], axis=1, keepdims=True)
    out_ref[0, :, :] = acc + bsum


def _tc_matmul(q, w_t, b_t):
    grid = (BATCH, TLEN // TB)
    return pl.pallas_call(
        _tc_matmul_body,
        grid=grid,
        in_specs=[
            pl.BlockSpec((TB, KDIM), lambda bi, ti: (bi * (TLEN // TB) + ti, 0)),
            pl.BlockSpec((HIDDEN, KDIM), lambda bi, ti: (0, 0)),
            pl.BlockSpec((HIDDEN, NUM_Q), lambda bi, ti: (0, 0)),
        ],
        out_specs=pl.BlockSpec((1, HIDDEN, TB), lambda bi, ti: (bi, 0, ti)),
        out_shape=jax.ShapeDtypeStruct((BATCH, HIDDEN, TLEN), jnp.float32),
        compiler_params=pltpu.CompilerParams(
            dimension_semantics=("parallel", "parallel"),
        ),
    )(q, w_t, b_t)


def kernel(codes, codebooks, W, b):
    codes_flat = codes.astype(jnp.int32).reshape(NUM_Q, NTOK)
    cb_flat = codebooks.reshape(NUM_Q * CODEBOOK_SIZE, DIM).astype(jnp.bfloat16)
    w_t = jnp.transpose(W.reshape(KDIM, HIDDEN)).astype(jnp.bfloat16)
    b_t = jnp.transpose(b)
    q = _sc_gather(codes_flat, cb_flat)
    return _tc_matmul(q, w_t, b_t)


# speedup vs baseline: 1.4961x; 1.4961x over previous
import functools

import jax
import jax.numpy as jnp
from jax import lax
from jax.experimental import pallas as pl
from jax.experimental.pallas import tpu as pltpu
from jax.experimental.pallas import tpu_sc as plsc

NUM_Q = 8
CODEBOOK_SIZE = 1024
DIM = 64
HIDDEN = 1024
BATCH = 16
TLEN = 2048
NTOK = BATCH * TLEN
KDIM = NUM_Q * DIM
NPAIR = KDIM // 128

NC = 2
NS = 16
NW = NC * NS
TOK_PER_W = NTOK // NW
CHUNK = 64
NCHUNK = TOK_PER_W // CHUNK


def _sc_gather(codes_flat, cb_flat):
    mesh = plsc.VectorSubcoreMesh(
        core_axis_name="c", subcore_axis_name="s", num_cores=NC, num_subcores=NS
    )

    @functools.partial(
        pl.kernel,
        mesh=mesh,
        out_type=jax.ShapeDtypeStruct((NPAIR, NTOK, 2 * DIM), jnp.float32),
        scratch_types=[
            pltpu.VMEM((NUM_Q, CHUNK), jnp.int32),
            pltpu.VMEM((NUM_Q, CHUNK, DIM), jnp.float32),
            pltpu.SemaphoreType.DMA,
        ],
        compiler_params=pltpu.CompilerParams(use_tc_tiling_on_sc=False),
    )
    def k(codes_hbm, cb_hbm, q_hbm, idx_v, dst_v, sem):
        wid = lax.axis_index("s") * NC + lax.axis_index("c")
        wbase = wid * TOK_PER_W

        def chunk_body(ci, carry):
            base = wbase + ci * CHUNK
            pltpu.sync_copy(codes_hbm.at[:, pl.ds(base, CHUNK)], idx_v)
            for i in range(1, NUM_Q):
                for j in range(CHUNK // 16):
                    sl = pl.ds(j * 16, 16)
                    idx_v[i, sl] = idx_v[i, sl] + (i * CODEBOOK_SIZE)
            copies = [
                pltpu.async_copy(
                    cb_hbm.at[idx_v.at[i]],
                    dst_v.at[i],
                    sem,
                )
                for i in range(NUM_Q)
            ]
            for cp in copies:
                cp.wait()
            for i in range(NUM_Q):
                pltpu.sync_copy(
                    dst_v.at[i],
                    q_hbm.at[i // 2, pl.ds(base, CHUNK), pl.ds((i % 2) * DIM, DIM)],
                )
            return carry

        lax.fori_loop(0, NCHUNK, chunk_body, 0)

    return k(codes_flat, cb_flat)


TB = 512


def _tc_matmul_body(q_ref, wt_ref, bt_ref, out_ref):
    acc = None
    for j in range(NPAIR):
        part = lax.dot_general(
            wt_ref[j], q_ref[j],
            dimension_numbers=(((1,), (1,)), ((), ())),
            preferred_element_type=jnp.float32,
        )
        acc = part if acc is None else acc + part
    bsum = jnp.sum(bt_ref[...], axis=1, keepdims=True)
    out_ref[0, :, :] = acc + bsum


def _tc_matmul(q, w_t, b_t):
    grid = (BATCH, TLEN // TB)
    return pl.pallas_call(
        _tc_matmul_body,
        grid=grid,
        in_specs=[
            pl.BlockSpec(
                (NPAIR, TB, 2 * DIM), lambda bi, ti: (0, bi * (TLEN // TB) + ti, 0)
            ),
            pl.BlockSpec((NPAIR, HIDDEN, 2 * DIM), lambda bi, ti: (0, 0, 0)),
            pl.BlockSpec((HIDDEN, NUM_Q), lambda bi, ti: (0, 0)),
        ],
        out_specs=pl.BlockSpec((1, HIDDEN, TB), lambda bi, ti: (bi, 0, ti)),
        out_shape=jax.ShapeDtypeStruct((BATCH, HIDDEN, TLEN), jnp.float32),
        compiler_params=pltpu.CompilerParams(
            dimension_semantics=("parallel", "parallel"),
        ),
    )(q, w_t, b_t)


def kernel(codes, codebooks, W, b):
    codes_flat = codes.astype(jnp.int32).reshape(NUM_Q, NTOK)
    cb_flat = codebooks.reshape(NUM_Q * CODEBOOK_SIZE, DIM)
    w_t = (
        jnp.transpose(W.reshape(KDIM, HIDDEN))
        .reshape(HIDDEN, NPAIR, 2 * DIM)
        .transpose(1, 0, 2)
        .astype(jnp.bfloat16)
    )
    b_t = jnp.transpose(b)
    q = _sc_gather(codes_flat, cb_flat)
    return _tc_matmul(q, w_t, b_t)


# speedup vs baseline: 1.5948x; 1.0659x over previous
import functools

import jax
import jax.numpy as jnp
from jax import lax
from jax.experimental import pallas as pl
from jax.experimental.pallas import tpu as pltpu
from jax.experimental.pallas import tpu_sc as plsc

NUM_Q = 8
CODEBOOK_SIZE = 1024
DIM = 64
HIDDEN = 1024
BATCH = 16
TLEN = 2048
NTOK = BATCH * TLEN
KDIM = NUM_Q * DIM
NPAIR = KDIM // 128

NC = 2
NS = 16
NW = NC * NS
TOK_PER_W = NTOK // NW
CHUNK = 64
NCHUNK = TOK_PER_W // CHUNK


def _sc_gather(codes_flat, cb_flat):
    mesh = plsc.VectorSubcoreMesh(
        core_axis_name="c", subcore_axis_name="s", num_cores=NC, num_subcores=NS
    )

    @functools.partial(
        pl.kernel,
        mesh=mesh,
        out_type=jax.ShapeDtypeStruct((NPAIR, NTOK, 2 * DIM), jnp.float32),
        scratch_types=[
            pltpu.VMEM((NUM_Q, CHUNK), jnp.int32),
            pltpu.VMEM((NUM_Q, CHUNK, DIM), jnp.float32),
            pltpu.SemaphoreType.DMA,
        ],
        compiler_params=pltpu.CompilerParams(use_tc_tiling_on_sc=False),
    )
    def k(codes_hbm, cb_hbm, q_hbm, idx_v, dst_v, sem):
        wid = lax.axis_index("s") * NC + lax.axis_index("c")
        wbase = wid * TOK_PER_W

        def chunk_body(ci, carry):
            base = wbase + ci * CHUNK
            pltpu.sync_copy(codes_hbm.at[:, pl.ds(base, CHUNK)], idx_v)
            for i in range(1, NUM_Q):
                for j in range(CHUNK // 16):
                    sl = pl.ds(j * 16, 16)
                    idx_v[i, sl] = idx_v[i, sl] + (i * CODEBOOK_SIZE)
            copies = [
                pltpu.async_copy(
                    cb_hbm.at[idx_v.at[i]],
                    dst_v.at[i],
                    sem,
                )
                for i in range(NUM_Q)
            ]
            for cp in copies:
                cp.wait()
            for i in range(NUM_Q):
                pltpu.sync_copy(
                    dst_v.at[i],
                    q_hbm.at[i // 2, pl.ds(base, CHUNK), pl.ds((i % 2) * DIM, DIM)],
                )
            return carry

        lax.fori_loop(0, NCHUNK, chunk_body, 0)

    return k(codes_flat, cb_flat)


TB = 512


def _tc_matmul_body(q_ref, wt_ref, bt_ref, out_ref):
    qb = jnp.concatenate([q_ref[j] for j in range(NPAIR)], axis=1)
    wb = jnp.concatenate([wt_ref[j] for j in range(NPAIR)], axis=1)
    acc = lax.dot_general(
        wb, qb,
        dimension_numbers=(((1,), (1,)), ((), ())),
        preferred_element_type=jnp.float32,
    )
    bsum = jnp.sum(bt_ref[...], axis=1, keepdims=True)
    out_ref[0, :, :] = acc + bsum


def _tc_matmul(q, w_t, b_t):
    grid = (BATCH, TLEN // TB)
    return pl.pallas_call(
        _tc_matmul_body,
        grid=grid,
        in_specs=[
            pl.BlockSpec(
                (NPAIR, TB, 2 * DIM), lambda bi, ti: (0, bi * (TLEN // TB) + ti, 0)
            ),
            pl.BlockSpec((NPAIR, HIDDEN, 2 * DIM), lambda bi, ti: (0, 0, 0)),
            pl.BlockSpec((HIDDEN, NUM_Q), lambda bi, ti: (0, 0)),
        ],
        out_specs=pl.BlockSpec((1, HIDDEN, TB), lambda bi, ti: (bi, 0, ti)),
        out_shape=jax.ShapeDtypeStruct((BATCH, HIDDEN, TLEN), jnp.float32),
        compiler_params=pltpu.CompilerParams(
            dimension_semantics=("parallel", "parallel"),
        ),
    )(q, w_t, b_t)


def kernel(codes, codebooks, W, b):
    codes_flat = codes.astype(jnp.int32).reshape(NUM_Q, NTOK)
    cb_flat = codebooks.reshape(NUM_Q * CODEBOOK_SIZE, DIM)
    w_t = (
        jnp.transpose(W.reshape(KDIM, HIDDEN))
        .reshape(HIDDEN, NPAIR, 2 * DIM)
        .transpose(1, 0, 2)
        .astype(jnp.bfloat16)
    )
    b_t = jnp.transpose(b)
    q = _sc_gather(codes_flat, cb_flat)
    return _tc_matmul(q, w_t, b_t)


# speedup vs baseline: 1.8332x; 1.1495x over previous
import functools

import jax
import jax.numpy as jnp
from jax import lax
from jax.experimental import pallas as pl
from jax.experimental.pallas import tpu as pltpu
from jax.experimental.pallas import tpu_sc as plsc

NUM_Q = 8
CODEBOOK_SIZE = 1024
DIM = 64
HIDDEN = 1024
BATCH = 16
TLEN = 2048
NTOK = BATCH * TLEN
KDIM = NUM_Q * DIM
NPAIR = KDIM // 128

NC = 2
NS = 16
NW = NC * NS
NSLICE = 4
BSLICE = BATCH // NSLICE
NTOK_S = NTOK // NSLICE
TOK_PER_W = NTOK_S // NW
CHUNK = 64
NCHUNK = TOK_PER_W // CHUNK


def _sc_gather(codes_flat, cb_flat):
    mesh = plsc.VectorSubcoreMesh(
        core_axis_name="c", subcore_axis_name="s", num_cores=NC, num_subcores=NS
    )

    @functools.partial(
        pl.kernel,
        mesh=mesh,
        out_type=jax.ShapeDtypeStruct((NPAIR, NTOK_S, 2 * DIM), jnp.float32),
        scratch_types=[
            pltpu.VMEM((NUM_Q, CHUNK), jnp.int32),
            pltpu.VMEM((NUM_Q, CHUNK, DIM), jnp.float32),
            pltpu.SemaphoreType.DMA,
        ],
        compiler_params=pltpu.CompilerParams(use_tc_tiling_on_sc=False),
    )
    def k(codes_hbm, cb_hbm, q_hbm, idx_v, dst_v, sem):
        wid = lax.axis_index("s") * NC + lax.axis_index("c")
        wbase = wid * TOK_PER_W

        def chunk_body(ci, carry):
            base = wbase + ci * CHUNK
            pltpu.sync_copy(codes_hbm.at[:, pl.ds(base, CHUNK)], idx_v)
            for i in range(1, NUM_Q):
                for j in range(CHUNK // 16):
                    sl = pl.ds(j * 16, 16)
                    idx_v[i, sl] = idx_v[i, sl] + (i * CODEBOOK_SIZE)
            copies = [
                pltpu.async_copy(
                    cb_hbm.at[idx_v.at[i]],
                    dst_v.at[i],
                    sem,
                )
                for i in range(NUM_Q)
            ]
            for cp in copies:
                cp.wait()
            for i in range(NUM_Q):
                pltpu.sync_copy(
                    dst_v.at[i],
                    q_hbm.at[i // 2, pl.ds(base, CHUNK), pl.ds((i % 2) * DIM, DIM)],
                )
            return carry

        lax.fori_loop(0, NCHUNK, chunk_body, 0)

    return k(codes_flat, cb_flat)


TB = 512


def _tc_matmul_body(q_ref, wt_ref, bt_ref, out_ref):
    qb = jnp.concatenate([q_ref[j] for j in range(NPAIR)], axis=1)
    wb = jnp.concatenate([wt_ref[j] for j in range(NPAIR)], axis=1)
    acc = lax.dot_general(
        wb, qb,
        dimension_numbers=(((1,), (1,)), ((), ())),
        preferred_element_type=jnp.float32,
    )
    bsum = jnp.sum(bt_ref[...], axis=1, keepdims=True)
    out_ref[0, :, :] = acc + bsum


def _tc_matmul(carry, q, w_t, b_t, bo):
    grid = (BSLICE, TLEN // TB)
    body = _tc_matmul_body
    in_specs = [
        pl.BlockSpec(
            (NPAIR, TB, 2 * DIM), lambda bi, ti: (0, bi * (TLEN // TB) + ti, 0)
        ),
        pl.BlockSpec((NPAIR, HIDDEN, 2 * DIM), lambda bi, ti: (0, 0, 0)),
        pl.BlockSpec((HIDDEN, NUM_Q), lambda bi, ti: (0, 0)),
    ]
    args = (q, w_t, b_t)
    aliases = {}
    if carry is not None:
        body = lambda c_ref, q_ref, wt_ref, bt_ref, out_ref: _tc_matmul_body(
            q_ref, wt_ref, bt_ref, out_ref
        )
        in_specs = [pl.BlockSpec(memory_space=pltpu.MemorySpace.HBM)] + in_specs
        args = (carry,) + args
        aliases = {0: 0}
    return pl.pallas_call(
        body,
        grid=grid,
        in_specs=in_specs,
        out_specs=pl.BlockSpec((1, HIDDEN, TB), lambda bi, ti: (bo + bi, 0, ti)),
        out_shape=jax.ShapeDtypeStruct((BATCH, HIDDEN, TLEN), jnp.float32),
        input_output_aliases=aliases,
        compiler_params=pltpu.CompilerParams(
            dimension_semantics=("arbitrary", "arbitrary"),
        ),
    )(*args)


def kernel(codes, codebooks, W, b):
    codes_flat = codes.astype(jnp.int32).reshape(NUM_Q, NTOK)
    cb_flat = codebooks.reshape(NUM_Q * CODEBOOK_SIZE, DIM)
    w_t = (
        jnp.transpose(W.reshape(KDIM, HIDDEN))
        .reshape(HIDDEN, NPAIR, 2 * DIM)
        .transpose(1, 0, 2)
        .astype(jnp.bfloat16)
    )
    b_t = jnp.transpose(b)
    qs = [
        _sc_gather(
            lax.slice_in_dim(codes_flat, s * NTOK_S, (s + 1) * NTOK_S, axis=1),
            cb_flat,
        )
        for s in range(NSLICE)
    ]
    out = None
    for s in range(NSLICE):
        out = _tc_matmul(out, qs[s], w_t, b_t, s * BSLICE)
    return out


# speedup vs baseline: 1.9192x; 1.0469x over previous
import functools

import jax
import jax.numpy as jnp
from jax import lax
from jax.experimental import pallas as pl
from jax.experimental.pallas import tpu as pltpu
from jax.experimental.pallas import tpu_sc as plsc

NUM_Q = 8
CODEBOOK_SIZE = 1024
DIM = 64
HIDDEN = 1024
BATCH = 16
TLEN = 2048
NTOK = BATCH * TLEN
KDIM = NUM_Q * DIM
NPAIR = KDIM // 128

NC = 2
NS = 16
NW = NC * NS
NSLICE = 4
BSLICE = BATCH // NSLICE
NTOK_S = NTOK // NSLICE
TOK_PER_W = NTOK_S // NW
CHUNK = 128
NCHUNK = TOK_PER_W // CHUNK


def _sc_gather(codes_flat, cb_flat):
    mesh = plsc.VectorSubcoreMesh(
        core_axis_name="c", subcore_axis_name="s", num_cores=NC, num_subcores=NS
    )

    @functools.partial(
        pl.kernel,
        mesh=mesh,
        out_type=jax.ShapeDtypeStruct((NPAIR, NTOK_S, 2 * DIM), jnp.float32),
        scratch_types=[
            pltpu.VMEM((NUM_Q, CHUNK), jnp.int32),
            pltpu.VMEM((NUM_Q, CHUNK, DIM), jnp.float32),
            pltpu.SemaphoreType.DMA,
        ],
        compiler_params=pltpu.CompilerParams(use_tc_tiling_on_sc=False),
    )
    def k(codes_hbm, cb_hbm, q_hbm, idx_v, dst_v, sem):
        wid = lax.axis_index("s") * NC + lax.axis_index("c")
        wbase = wid * TOK_PER_W

        def chunk_body(ci, carry):
            base = wbase + ci * CHUNK
            pltpu.sync_copy(codes_hbm.at[:, pl.ds(base, CHUNK)], idx_v)
            for i in range(1, NUM_Q):
                for j in range(CHUNK // 16):
                    sl = pl.ds(j * 16, 16)
                    idx_v[i, sl] = idx_v[i, sl] + (i * CODEBOOK_SIZE)
            copies = [
                pltpu.async_copy(
                    cb_hbm.at[idx_v.at[i]],
                    dst_v.at[i],
                    sem,
                )
                for i in range(NUM_Q)
            ]
            for cp in copies:
                cp.wait()
            for i in range(NUM_Q):
                pltpu.sync_copy(
                    dst_v.at[i],
                    q_hbm.at[i // 2, pl.ds(base, CHUNK), pl.ds((i % 2) * DIM, DIM)],
                )
            return carry

        lax.fori_loop(0, NCHUNK, chunk_body, 0)

    return k(codes_flat, cb_flat)


TB = 1024


def _tc_matmul_body(q_ref, wt_ref, bt_ref, out_ref):
    qb = jnp.concatenate([q_ref[j] for j in range(NPAIR)], axis=1)
    wb = jnp.concatenate([wt_ref[j] for j in range(NPAIR)], axis=1)
    acc = lax.dot_general(
        wb, qb,
        dimension_numbers=(((1,), (1,)), ((), ())),
        preferred_element_type=jnp.float32,
    )
    bsum = jnp.sum(bt_ref[...], axis=1, keepdims=True)
    out_ref[0, :, :] = acc + bsum


def _tc_matmul(carry, q, w_t, b_t, bo):
    grid = (BSLICE, TLEN // TB)
    body = _tc_matmul_body
    in_specs = [
        pl.BlockSpec(
            (NPAIR, TB, 2 * DIM), lambda bi, ti: (0, bi * (TLEN // TB) + ti, 0)
        ),
        pl.BlockSpec((NPAIR, HIDDEN, 2 * DIM), lambda bi, ti: (0, 0, 0)),
        pl.BlockSpec((HIDDEN, NUM_Q), lambda bi, ti: (0, 0)),
    ]
    args = (q, w_t, b_t)
    aliases = {}
    if carry is not None:
        body = lambda c_ref, q_ref, wt_ref, bt_ref, out_ref: _tc_matmul_body(
            q_ref, wt_ref, bt_ref, out_ref
        )
        in_specs = [pl.BlockSpec(memory_space=pltpu.MemorySpace.HBM)] + in_specs
        args = (carry,) + args
        aliases = {0: 0}
    return pl.pallas_call(
        body,
        grid=grid,
        in_specs=in_specs,
        out_specs=pl.BlockSpec((1, HIDDEN, TB), lambda bi, ti: (bo + bi, 0, ti)),
        out_shape=jax.ShapeDtypeStruct((BATCH, HIDDEN, TLEN), jnp.float32),
        input_output_aliases=aliases,
        compiler_params=pltpu.CompilerParams(
            dimension_semantics=("arbitrary", "arbitrary"),
        ),
    )(*args)


def kernel(codes, codebooks, W, b):
    codes_flat = codes.astype(jnp.int32).reshape(NUM_Q, NTOK)
    cb_flat = codebooks.reshape(NUM_Q * CODEBOOK_SIZE, DIM)
    w_t = (
        jnp.transpose(W.reshape(KDIM, HIDDEN))
        .reshape(HIDDEN, NPAIR, 2 * DIM)
        .transpose(1, 0, 2)
        .astype(jnp.bfloat16)
    )
    b_t = jnp.transpose(b)
    qs = [
        _sc_gather(
            lax.slice_in_dim(codes_flat, s * NTOK_S, (s + 1) * NTOK_S, axis=1),
            cb_flat,
        )
        for s in range(NSLICE)
    ]
    out = None
    for s in range(NSLICE):
        out = _tc_matmul(out, qs[s], w_t, b_t, s * BSLICE)
    return out
